# baseline (speedup 1.0000x reference)
import functools

import jax
import jax.numpy as jnp
from jax import lax
from jax.experimental import pallas as pl
from jax.experimental.pallas import tpu as pltpu
from jax.experimental.pallas import tpu_sc as plsc

B = 16
N = 2048
KNN = 16
R = 32
RC = 32


def _knn_body(xall_ref, xblk_ref, idx_ref, work_ref, fiota_ref):
    b = pl.program_id(0)
    xall = xall_ref[0]
    xblk = xblk_ref[0]
    xx_all = jnp.sum(xall * xall, axis=1)
    xx_blk = jnp.sum(xblk * xblk, axis=1)
    inner = -2.0 * lax.dot_general(
        xblk, xall, (((1,), (1,)), ((), ())),
        preferred_element_type=jnp.float32)
    work_ref[...] = (-xx_all[None, :] - inner) - xx_blk[:, None]
    fiota_ref[...] = lax.broadcasted_iota(
        jnp.int32, (R, N), 1).astype(jnp.float32)
    kiota = lax.broadcasted_iota(jnp.int32, (R, KNN), 1)
    neg = jnp.float32(-3e38)
    bigf = jnp.float32(N)

    def rowmax(v):
        w = jnp.maximum(v[:, :1024], v[:, 1024:])
        w = jnp.maximum(w[:, :512], w[:, 512:])
        w = jnp.maximum(w[:, :256], w[:, 256:512])
        m2 = jnp.max(w.reshape(R, 2, 128), axis=2)
        return jnp.max(m2, axis=1, keepdims=True)

    m0 = rowmax(work_ref[...])

    def step(carry):
        k, idx16, m = carry
        cand = fiota_ref[...] - bigf * jnp.sign(work_ref[...] - m)
        c = jnp.minimum(cand[:, :1024], cand[:, 1024:])
        c = jnp.minimum(c[:, :512], c[:, 512:])
        c = jnp.minimum(c[:, :256], c[:, 256:512])
        a2 = jnp.min(c.reshape(R, 2, 128), axis=2)
        am = jnp.min(a2, axis=1, keepdims=True)
        idx16 = jnp.where(kiota == k, am.astype(jnp.int32), idx16)
        masked = work_ref[...] + (
            1.0 - jnp.minimum(jnp.abs(fiota_ref[...] - am), 1.0)) * neg
        work_ref[...] = masked
        return k + 1, idx16, rowmax(masked)

    _, idx16, _ = lax.while_loop(
        lambda c: c[0] < KNN, step,
        (jnp.int32(0), jnp.zeros((R, KNN), jnp.int32), m0))
    idx_ref[0] = idx16 + b * N


def _knn(pts):
    nb = N // R
    return pl.pallas_call(
        _knn_body,
        grid=(B, nb),
        in_specs=[
            pl.BlockSpec((1, N, 2), lambda b, rb: (b, 0, 0)),
            pl.BlockSpec((1, R, 2), lambda b, rb: (b, rb, 0)),
        ],
        out_specs=pl.BlockSpec((1, R, KNN), lambda b, rb: (b, rb, 0)),
        out_shape=jax.ShapeDtypeStruct((B, N, KNN), jnp.int32),
        scratch_shapes=[
            pltpu.VMEM((R, N), jnp.float32),
            pltpu.VMEM((R, N), jnp.float32),
        ],
    )(pts, pts)


def _cov_mlp1_body(xblk_ref, g0_ref, g1_ref, w1a_ref, b1a_ref, w1b_ref,
                   b1b_ref, w1c_ref, b1c_ref, f1_ref):
    xblk = xblk_ref[...]
    g0 = g0_ref[:, 0:2]
    g1 = g1_ref[:, 0:2]
    cov = jnp.concatenate([
        (g0[:, 0] * g1[:, 0])[:, None],
        (g0[:, 0] * g1[:, 1])[:, None],
        (g0[:, 1] * g1[:, 0])[:, None],
        (g0[:, 1] * g1[:, 1])[:, None],
    ], axis=1)
    h8 = jnp.concatenate([xblk, cov, jnp.zeros((RC, 2), jnp.float32)],
                         axis=1)
    a = jnp.maximum(
        jnp.dot(h8, w1a_ref[...], preferred_element_type=jnp.float32)
        + b1a_ref[...], 0.0)
    a = jnp.maximum(
        jnp.dot(a, w1b_ref[...], preferred_element_type=jnp.float32)
        + b1b_ref[...], 0.0)
    a = jnp.maximum(
        jnp.dot(a, w1c_ref[...], preferred_element_type=jnp.float32)
        + b1c_ref[...], 0.0)
    f1_ref[...] = jnp.concatenate([a, jnp.zeros((RC, 64), jnp.float32)],
                                  axis=1)


def _cov_mlp1(pts, g0, g1, w1aT, b1a, w1bT, b1b, w1cT, b1c):
    M = B * N
    wspec = pl.BlockSpec((8, 64), lambda i: (0, 0))
    bspec = pl.BlockSpec((1, 64), lambda i: (0, 0))
    w64spec = pl.BlockSpec((64, 64), lambda i: (0, 0))
    return pl.pallas_call(
        _cov_mlp1_body,
        grid=(M // RC,),
        in_specs=[
            pl.BlockSpec((RC, 2), lambda i: (i, 0)),
            pl.BlockSpec((RC, 128), lambda i: (i, 0)),
            pl.BlockSpec((RC, 128), lambda i: (i, 0)),
            wspec, bspec, w64spec, bspec, w64spec, bspec,
        ],
        out_specs=pl.BlockSpec((RC, 128), lambda i: (i, 0)),
        out_shape=jax.ShapeDtypeStruct((M, 128), jnp.float32),
    )(pts, g0, g1, w1aT, b1a, w1bT, b1b, w1cT, b1c)


def _make_lmax(C):
    NC, NS = 2, 16
    NW = NC * NS
    M = B * N
    per_w = M // NW
    P = 8
    G = P * KNN
    nch = per_w // P
    mesh = plsc.VectorSubcoreMesh(core_axis_name="c", subcore_axis_name="s")

    @functools.partial(
        pl.kernel, mesh=mesh,
        out_type=jax.ShapeDtypeStruct((M, C), jnp.float32),
        scratch_types=[
            pltpu.VMEM((G,), jnp.int32),
            pltpu.VMEM((G, C), jnp.float32),
            pltpu.VMEM((P, C), jnp.float32),
            pltpu.SemaphoreType.DMA,
        ],
    )
    def kern(table, idxf, out, idx_v, rows_v, out_v, sem):
        cid = lax.axis_index("c")
        sid = lax.axis_index("s")
        wid = sid * NC + cid
        base_pt = wid * per_w

        def chunk(g, carry):
            pt0 = base_pt + g * P
            pltpu.sync_copy(idxf.at[pl.ds(pt0 * KNN, G)], idx_v)
            pltpu.async_copy(table.at[idx_v], rows_v, sem).wait()
            for p in range(P):
                for cc in range(C // 16):
                    sl = pl.ds(cc * 16, 16)
                    acc = rows_v[p * KNN, sl]
                    for r in range(1, KNN):
                        acc = jnp.maximum(acc, rows_v[p * KNN + r, sl])
                    out_v[p, sl] = acc
            pltpu.sync_copy(out_v, out.at[pl.ds(pt0, P)])
            return carry

        lax.fori_loop(0, nch, chunk, 0)

    return kern


def _make_gather():
    NC, NS = 2, 16
    NW = NC * NS
    M = B * N
    per_w = M // NW
    P = 256
    nch = per_w // P
    mesh = plsc.VectorSubcoreMesh(core_axis_name="c", subcore_axis_name="s")

    @functools.partial(
        pl.kernel, mesh=mesh,
        out_type=jax.ShapeDtypeStruct((M, 128), jnp.float32),
        scratch_types=[
            pltpu.VMEM((P,), jnp.int32),
            pltpu.VMEM((P, 128), jnp.float32),
            pltpu.SemaphoreType.DMA,
        ],
    )
    def kern(table, idxc, out, idx_v, rows_v, sem):
        cid = lax.axis_index("c")
        sid = lax.axis_index("s")
        wid = sid * NC + cid
        base = wid * per_w

        def chunk(g, carry):
            r0 = base + g * P
            pltpu.sync_copy(idxc.at[pl.ds(r0, P)], idx_v)
            pltpu.async_copy(table.at[idx_v], rows_v, sem).wait()
            pltpu.sync_copy(rows_v, out.at[pl.ds(r0, P)])
            return carry

        lax.fori_loop(0, nch, chunk, 0)

    return kern


@functools.cache
def _lmax(C):
    return _make_lmax(C)


@functools.cache
def _gather():
    return _make_gather()


def _lin_body(m_ref, wl_ref, bl_ref, wc_ref, bc_ref, out_ref):
    t = jnp.dot(m_ref[...], wl_ref[...],
                preferred_element_type=jnp.float32) + bl_ref[...]
    out_ref[...] = jnp.maximum(
        jnp.dot(t, wc_ref[...], preferred_element_type=jnp.float32)
        + bc_ref[...], 0.0)


def _lin(m1, wl1T, bl1, wc1T, bc1):
    P = 2048
    M = B * N
    return pl.pallas_call(
        _lin_body,
        grid=(M // P,),
        in_specs=[
            pl.BlockSpec((P, 128), lambda i: (i, 0)),
            pl.BlockSpec((128, 64), lambda i: (0, 0)),
            pl.BlockSpec((1, 64), lambda i: (0, 0)),
            pl.BlockSpec((64, 128), lambda i: (0, 0)),
            pl.BlockSpec((1, 128), lambda i: (0, 0)),
        ],
        out_specs=pl.BlockSpec((P, 128), lambda i: (i, 0)),
        out_shape=jax.ShapeDtypeStruct((M, 128), jnp.float32),
    )(m1, wl1T, bl1, wc1T, bc1)


def _tail_body(m2_ref, wl2_ref, bl2_ref, wc2_ref, bc2_ref,
               wma_ref, bma_ref, wmb_ref, bmb_ref, out_ref):
    t = jnp.dot(m2_ref[0], wl2_ref[...],
                preferred_element_type=jnp.float32) + bl2_ref[...]
    u = jnp.dot(t, wc2_ref[...],
                preferred_element_type=jnp.float32) + bc2_ref[...]
    mx = jnp.max(u, axis=0, keepdims=True)
    a = jnp.maximum(
        jnp.dot(mx, wma_ref[...], preferred_element_type=jnp.float32)
        + bma_ref[...], 0.0)
    out_ref[0] = jnp.dot(a, wmb_ref[...],
                         preferred_element_type=jnp.float32) + bmb_ref[...]


def _tail(m2, wl2T, bl2, wc2T, bc2, wmaT, bma, wmbT, bmb):
    full = lambda b: (0, 0)
    return pl.pallas_call(
        _tail_body,
        grid=(B,),
        in_specs=[
            pl.BlockSpec((1, N, 128), lambda b: (b, 0, 0)),
            pl.BlockSpec((128, 128), full),
            pl.BlockSpec((1, 128), full),
            pl.BlockSpec((128, 1024), full),
            pl.BlockSpec((1, 1024), full),
            pl.BlockSpec((1024, 512), full),
            pl.BlockSpec((1, 512), full),
            pl.BlockSpec((512, 512), full),
            pl.BlockSpec((1, 512), full),
        ],
        out_specs=pl.BlockSpec((1, 1, 512), lambda b: (b, 0, 0)),
        out_shape=jax.ShapeDtypeStruct((B, 1, 512), jnp.float32),
    )(m2, wl2T, bl2, wc2T, bc2, wmaT, bma, wmbT, bmb)


def kernel(pts, W1a, b1a, W1b, b1b, W1c, b1c, Wl1, bl1, Wc1, bc1,
           Wl2, bl2, Wc2, bc2, Wm2a, bm2a, Wm2b, bm2b):
    w1aT = jnp.pad(W1a, ((0, 0), (0, 2))).T
    idx = _knn(pts)
    ptsf = pts.reshape(B * N, 2)
    pts_pad = jnp.pad(ptsf, ((0, 0), (0, 126)))
    g0 = _gather()(pts_pad, idx[:, :, 0].reshape(-1))
    g1 = _gather()(pts_pad, idx[:, :, 1].reshape(-1))
    f1 = _cov_mlp1(ptsf, g0, g1, w1aT, b1a[None, :], W1b.T, b1b[None, :],
                   W1c.T, b1c[None, :])
    idxf = idx.reshape(-1)
    wl1T = jnp.pad(Wl1.T, ((0, 64), (0, 0)))
    m1 = _lmax(128)(f1, idxf)
    f2 = _lin(m1, wl1T, bl1[None, :], Wc1.T, bc1[None, :])
    m2 = _lmax(128)(f2, idxf)
    feat = _tail(m2.reshape(B, N, 128), Wl2.T, bl2[None, :], Wc2.T,
                 bc2[None, :], Wm2a.T, bm2a[None, :], Wm2b.T, bm2b[None, :])
    return feat


# speedup vs baseline: 3.9162x; 3.9162x over previous
import functools

import jax
import jax.numpy as jnp
from jax import lax
from jax.experimental import pallas as pl
from jax.experimental.pallas import tpu as pltpu
from jax.experimental.pallas import tpu_sc as plsc

B = 16
N = 2048
KNN = 16
R = 64
RC = 512


def _knn_body(xallT_ref, xblk_ref, idx_ref, work_ref, fiota_ref):
    b = pl.program_id(0)
    xallT = xallT_ref[0]
    xblk = xblk_ref[0]
    xx_all = jnp.sum(xallT * xallT, axis=0, keepdims=True)
    xx_blk = jnp.sum(xblk * xblk, axis=1)
    inner = -2.0 * lax.dot_general(
        xblk, xallT, (((1,), (0,)), ((), ())),
        preferred_element_type=jnp.float32)
    work_ref[...] = (-xx_all - inner) - xx_blk[:, None]
    fiota_ref[...] = lax.broadcasted_iota(
        jnp.int32, (R, N), 1).astype(jnp.float32)
    kiota = lax.broadcasted_iota(jnp.int32, (R, KNN), 1)
    neg = jnp.float32(-3e38)
    bigf = jnp.float32(N)

    def rowmax(v):
        w = jnp.maximum(v[:, :1024], v[:, 1024:])
        w = jnp.maximum(w[:, :512], w[:, 512:])
        w = jnp.maximum(w[:, :256], w[:, 256:512])
        m2 = jnp.max(w.reshape(R, 2, 128), axis=2)
        return jnp.max(m2, axis=1, keepdims=True)

    m0 = rowmax(work_ref[...])

    def step(carry):
        k, idx16, m = carry
        cand = fiota_ref[...] - bigf * jnp.sign(work_ref[...] - m)
        c = jnp.minimum(cand[:, :1024], cand[:, 1024:])
        c = jnp.minimum(c[:, :512], c[:, 512:])
        c = jnp.minimum(c[:, :256], c[:, 256:512])
        a2 = jnp.min(c.reshape(R, 2, 128), axis=2)
        am = jnp.min(a2, axis=1, keepdims=True)
        idx16 = jnp.where(kiota == k, am.astype(jnp.int32), idx16)
        masked = work_ref[...] + (
            1.0 - jnp.minimum(jnp.abs(fiota_ref[...] - am), 1.0)) * neg
        work_ref[...] = masked
        return k + 1, idx16, rowmax(masked)

    _, idx16, _ = lax.while_loop(
        lambda c: c[0] < KNN, step,
        (jnp.int32(0), jnp.zeros((R, KNN), jnp.int32), m0))
    idx_ref[0] = idx16 + b * N


def _knn(pts, ptsT):
    nb = N // R
    return pl.pallas_call(
        _knn_body,
        grid=(B, nb),
        in_specs=[
            pl.BlockSpec((1, 2, N), lambda b, rb: (b, 0, 0)),
            pl.BlockSpec((1, R, 2), lambda b, rb: (b, rb, 0)),
        ],
        out_specs=pl.BlockSpec((1, R, KNN), lambda b, rb: (b, rb, 0)),
        out_shape=jax.ShapeDtypeStruct((B, N, KNN), jnp.int32),
        scratch_shapes=[
            pltpu.VMEM((R, N), jnp.float32),
            pltpu.VMEM((R, N), jnp.float32),
        ],
        compiler_params=pltpu.CompilerParams(
            vmem_limit_bytes=100 * 1024 * 1024),
    )(ptsT, pts)


def _cov_mlp1_body(xblk_ref, g0_ref, g1_ref, w1a_ref, b1a_ref, w1b_ref,
                   b1b_ref, w1c_ref, b1c_ref, f1_ref):
    xblk = xblk_ref[...]
    g0 = g0_ref[:, 0:2]
    g1 = g1_ref[:, 0:2]
    cov = jnp.concatenate([
        (g0[:, 0] * g1[:, 0])[:, None],
        (g0[:, 0] * g1[:, 1])[:, None],
        (g0[:, 1] * g1[:, 0])[:, None],
        (g0[:, 1] * g1[:, 1])[:, None],
    ], axis=1)
    h8 = jnp.concatenate([xblk, cov, jnp.zeros((RC, 2), jnp.float32)],
                         axis=1)
    a = jnp.maximum(
        jnp.dot(h8, w1a_ref[...], preferred_element_type=jnp.float32)
        + b1a_ref[...], 0.0)
    a = jnp.maximum(
        jnp.dot(a, w1b_ref[...], preferred_element_type=jnp.float32)
        + b1b_ref[...], 0.0)
    a = jnp.maximum(
        jnp.dot(a, w1c_ref[...], preferred_element_type=jnp.float32)
        + b1c_ref[...], 0.0)
    f1_ref[...] = jnp.concatenate([a, jnp.zeros((RC, 64), jnp.float32)],
                                  axis=1)


def _cov_mlp1(pts, g0, g1, w1aT, b1a, w1bT, b1b, w1cT, b1c):
    M = B * N
    wspec = pl.BlockSpec((8, 64), lambda i: (0, 0))
    bspec = pl.BlockSpec((1, 64), lambda i: (0, 0))
    w64spec = pl.BlockSpec((64, 64), lambda i: (0, 0))
    return pl.pallas_call(
        _cov_mlp1_body,
        grid=(M // RC,),
        in_specs=[
            pl.BlockSpec((RC, 2), lambda i: (i, 0)),
            pl.BlockSpec((RC, 128), lambda i: (i, 0)),
            pl.BlockSpec((RC, 128), lambda i: (i, 0)),
            wspec, bspec, w64spec, bspec, w64spec, bspec,
        ],
        out_specs=pl.BlockSpec((RC, 128), lambda i: (i, 0)),
        out_shape=jax.ShapeDtypeStruct((M, 128), jnp.float32),
    )(pts, g0, g1, w1aT, b1a, w1bT, b1b, w1cT, b1c)


def _make_lmax(C):
    NC, NS = 2, 16
    NW = NC * NS
    M = B * N
    per_w = M // NW
    P = 8
    G = P * KNN
    nch = per_w // P
    mesh = plsc.VectorSubcoreMesh(core_axis_name="c", subcore_axis_name="s")

    @functools.partial(
        pl.kernel, mesh=mesh,
        out_type=jax.ShapeDtypeStruct((M, C), jnp.float32),
        scratch_types=[
            pltpu.VMEM((G,), jnp.int32),
            pltpu.VMEM((G, C), jnp.float32),
            pltpu.VMEM((P, C), jnp.float32),
            pltpu.SemaphoreType.DMA,
        ],
    )
    def kern(table, idxf, out, idx_v, rows_v, out_v, sem):
        cid = lax.axis_index("c")
        sid = lax.axis_index("s")
        wid = sid * NC + cid
        base_pt = wid * per_w

        def chunk(g, carry):
            pt0 = base_pt + g * P
            pltpu.sync_copy(idxf.at[pl.ds(pt0 * KNN, G)], idx_v)
            pltpu.async_copy(table.at[idx_v], rows_v, sem).wait()
            for p in range(P):
                for cc in range(C // 16):
                    sl = pl.ds(cc * 16, 16)
                    acc = rows_v[p * KNN, sl]
                    for r in range(1, KNN):
                        acc = jnp.maximum(acc, rows_v[p * KNN + r, sl])
                    out_v[p, sl] = acc
            pltpu.sync_copy(out_v, out.at[pl.ds(pt0, P)])
            return carry

        lax.fori_loop(0, nch, chunk, 0)

    return kern


def _make_gather():
    NC, NS = 2, 16
    NW = NC * NS
    M = B * N
    per_w = M // NW
    P = 256
    nch = per_w // P
    mesh = plsc.VectorSubcoreMesh(core_axis_name="c", subcore_axis_name="s")

    @functools.partial(
        pl.kernel, mesh=mesh,
        out_type=jax.ShapeDtypeStruct((M, 128), jnp.float32),
        scratch_types=[
            pltpu.VMEM((P,), jnp.int32),
            pltpu.VMEM((P, 128), jnp.float32),
            pltpu.SemaphoreType.DMA,
        ],
    )
    def kern(table, idxc, out, idx_v, rows_v, sem):
        cid = lax.axis_index("c")
        sid = lax.axis_index("s")
        wid = sid * NC + cid
        base = wid * per_w

        def chunk(g, carry):
            r0 = base + g * P
            pltpu.sync_copy(idxc.at[pl.ds(r0, P)], idx_v)
            pltpu.async_copy(table.at[idx_v], rows_v, sem).wait()
            pltpu.sync_copy(rows_v, out.at[pl.ds(r0, P)])
            return carry

        lax.fori_loop(0, nch, chunk, 0)

    return kern


@functools.cache
def _lmax(C):
    return _make_lmax(C)


@functools.cache
def _gather():
    return _make_gather()


def _lin_body(m_ref, wl_ref, bl_ref, wc_ref, bc_ref, out_ref):
    t = jnp.dot(m_ref[...], wl_ref[...],
                preferred_element_type=jnp.float32) + bl_ref[...]
    out_ref[...] = jnp.maximum(
        jnp.dot(t, wc_ref[...], preferred_element_type=jnp.float32)
        + bc_ref[...], 0.0)


def _lin(m1, wl1T, bl1, wc1T, bc1):
    P = 2048
    M = B * N
    return pl.pallas_call(
        _lin_body,
        grid=(M // P,),
        in_specs=[
            pl.BlockSpec((P, 128), lambda i: (i, 0)),
            pl.BlockSpec((128, 64), lambda i: (0, 0)),
            pl.BlockSpec((1, 64), lambda i: (0, 0)),
            pl.BlockSpec((64, 128), lambda i: (0, 0)),
            pl.BlockSpec((1, 128), lambda i: (0, 0)),
        ],
        out_specs=pl.BlockSpec((P, 128), lambda i: (i, 0)),
        out_shape=jax.ShapeDtypeStruct((M, 128), jnp.float32),
    )(m1, wl1T, bl1, wc1T, bc1)


def _tail_body(m2_ref, wl2_ref, bl2_ref, wc2_ref, bc2_ref,
               wma_ref, bma_ref, wmb_ref, bmb_ref, out_ref):
    t = jnp.dot(m2_ref[0], wl2_ref[...],
                preferred_element_type=jnp.float32) + bl2_ref[...]
    u = jnp.dot(t, wc2_ref[...],
                preferred_element_type=jnp.float32) + bc2_ref[...]
    mx = jnp.max(u, axis=0, keepdims=True)
    a = jnp.maximum(
        jnp.dot(mx, wma_ref[...], preferred_element_type=jnp.float32)
        + bma_ref[...], 0.0)
    out_ref[0] = jnp.dot(a, wmb_ref[...],
                         preferred_element_type=jnp.float32) + bmb_ref[...]


def _tail(m2, wl2T, bl2, wc2T, bc2, wmaT, bma, wmbT, bmb):
    full = lambda b: (0, 0)
    return pl.pallas_call(
        _tail_body,
        grid=(B,),
        in_specs=[
            pl.BlockSpec((1, N, 128), lambda b: (b, 0, 0)),
            pl.BlockSpec((128, 128), full),
            pl.BlockSpec((1, 128), full),
            pl.BlockSpec((128, 1024), full),
            pl.BlockSpec((1, 1024), full),
            pl.BlockSpec((1024, 512), full),
            pl.BlockSpec((1, 512), full),
            pl.BlockSpec((512, 512), full),
            pl.BlockSpec((1, 512), full),
        ],
        out_specs=pl.BlockSpec((1, 1, 512), lambda b: (b, 0, 0)),
        out_shape=jax.ShapeDtypeStruct((B, 1, 512), jnp.float32),
    )(m2, wl2T, bl2, wc2T, bc2, wmaT, bma, wmbT, bmb)


def kernel(pts, W1a, b1a, W1b, b1b, W1c, b1c, Wl1, bl1, Wc1, bc1,
           Wl2, bl2, Wc2, bc2, Wm2a, bm2a, Wm2b, bm2b):
    w1aT = jnp.pad(W1a, ((0, 0), (0, 2))).T
    ptsT = jnp.transpose(pts, (0, 2, 1))
    idx = _knn(pts, ptsT)
    ptsf = pts.reshape(B * N, 2)
    pts_pad = jnp.pad(ptsf, ((0, 0), (0, 126)))
    g0 = _gather()(pts_pad, idx[:, :, 0].reshape(-1))
    g1 = _gather()(pts_pad, idx[:, :, 1].reshape(-1))
    f1 = _cov_mlp1(ptsf, g0, g1, w1aT, b1a[None, :], W1b.T, b1b[None, :],
                   W1c.T, b1c[None, :])
    idxf = idx.reshape(-1)
    wl1T = jnp.pad(Wl1.T, ((0, 64), (0, 0)))
    m1 = _lmax(128)(f1, idxf)
    f2 = _lin(m1, wl1T, bl1[None, :], Wc1.T, bc1[None, :])
    m2 = _lmax(128)(f2, idxf)
    feat = _tail(m2.reshape(B, N, 128), Wl2.T, bl2[None, :], Wc2.T,
                 bc2[None, :], Wm2a.T, bm2a[None, :], Wm2b.T, bm2b[None, :])
    return feat


# speedup vs baseline: 4.0429x; 1.0323x over previous
import functools

import jax
import jax.numpy as jnp
from jax import lax
from jax.experimental import pallas as pl
from jax.experimental.pallas import tpu as pltpu
from jax.experimental.pallas import tpu_sc as plsc

B = 16
N = 2048
KNN = 16
R = 64
RC = 512


def _knn_body(xallT_ref, xblk_ref, idx_ref, work_ref, fiota_ref):
    b = pl.program_id(0)
    xallT = xallT_ref[0]
    xblk = xblk_ref[0]
    xx_all = jnp.sum(xallT * xallT, axis=0, keepdims=True)
    xx_blk = jnp.sum(xblk * xblk, axis=1)
    inner = -2.0 * lax.dot_general(
        xblk, xallT, (((1,), (0,)), ((), ())),
        preferred_element_type=jnp.float32)
    work_ref[...] = (-xx_all - inner) - xx_blk[:, None]
    fiota_ref[...] = lax.broadcasted_iota(
        jnp.int32, (R, N), 1).astype(jnp.float32)
    kiota = lax.broadcasted_iota(jnp.int32, (R, KNN), 1)
    neg = jnp.float32(-3e38)
    bigf = jnp.float32(N)

    def rowmax(v):
        w = jnp.maximum(v[:, :1024], v[:, 1024:])
        w = jnp.maximum(w[:, :512], w[:, 512:])
        w = jnp.maximum(w[:, :256], w[:, 256:512])
        m2 = jnp.max(w.reshape(R, 2, 128), axis=2)
        return jnp.max(m2, axis=1, keepdims=True)

    m0 = rowmax(work_ref[...])

    def step(carry):
        k, idx16, m = carry
        cand = fiota_ref[...] - bigf * jnp.sign(work_ref[...] - m)
        c = jnp.minimum(cand[:, :1024], cand[:, 1024:])
        c = jnp.minimum(c[:, :512], c[:, 512:])
        c = jnp.minimum(c[:, :256], c[:, 256:512])
        a2 = jnp.min(c.reshape(R, 2, 128), axis=2)
        am = jnp.min(a2, axis=1, keepdims=True)
        idx16 = jnp.where(kiota == k, am.astype(jnp.int32), idx16)
        masked = work_ref[...] + (
            1.0 - jnp.minimum(jnp.abs(fiota_ref[...] - am), 1.0)) * neg
        work_ref[...] = masked
        return k + 1, idx16, rowmax(masked)

    _, idx16, _ = lax.while_loop(
        lambda c: c[0] < KNN, step,
        (jnp.int32(0), jnp.zeros((R, KNN), jnp.int32), m0))
    idx_ref[0] = idx16 + b * N


def _knn(pts, ptsT):
    nb = N // R
    return pl.pallas_call(
        _knn_body,
        grid=(B, nb),
        in_specs=[
            pl.BlockSpec((1, 2, N), lambda b, rb: (b, 0, 0)),
            pl.BlockSpec((1, R, 2), lambda b, rb: (b, rb, 0)),
        ],
        out_specs=pl.BlockSpec((1, R, KNN), lambda b, rb: (b, rb, 0)),
        out_shape=jax.ShapeDtypeStruct((B, N, KNN), jnp.int32),
        scratch_shapes=[
            pltpu.VMEM((R, N), jnp.float32),
            pltpu.VMEM((R, N), jnp.float32),
        ],
        compiler_params=pltpu.CompilerParams(
            vmem_limit_bytes=100 * 1024 * 1024),
    )(ptsT, pts)


def _cov_mlp1_body(xblk_ref, g0_ref, g1_ref, w1a_ref, b1a_ref, w1b_ref,
                   b1b_ref, w1c_ref, b1c_ref, f1_ref):
    xblk = xblk_ref[...]
    g0 = g0_ref[:, 0:2]
    g1 = g1_ref[:, 0:2]
    cov = jnp.concatenate([
        (g0[:, 0] * g1[:, 0])[:, None],
        (g0[:, 0] * g1[:, 1])[:, None],
        (g0[:, 1] * g1[:, 0])[:, None],
        (g0[:, 1] * g1[:, 1])[:, None],
    ], axis=1)
    h8 = jnp.concatenate([xblk, cov, jnp.zeros((RC, 2), jnp.float32)],
                         axis=1)
    a = jnp.maximum(
        jnp.dot(h8, w1a_ref[...], preferred_element_type=jnp.float32)
        + b1a_ref[...], 0.0)
    a = jnp.maximum(
        jnp.dot(a, w1b_ref[...], preferred_element_type=jnp.float32)
        + b1b_ref[...], 0.0)
    a = jnp.maximum(
        jnp.dot(a, w1c_ref[...], preferred_element_type=jnp.float32)
        + b1c_ref[...], 0.0)
    f1_ref[...] = jnp.concatenate([a, jnp.zeros((RC, 64), jnp.float32)],
                                  axis=1)


def _cov_mlp1(pts, g0, g1, w1aT, b1a, w1bT, b1b, w1cT, b1c):
    M = B * N
    wspec = pl.BlockSpec((8, 64), lambda i: (0, 0))
    bspec = pl.BlockSpec((1, 64), lambda i: (0, 0))
    w64spec = pl.BlockSpec((64, 64), lambda i: (0, 0))
    return pl.pallas_call(
        _cov_mlp1_body,
        grid=(M // RC,),
        in_specs=[
            pl.BlockSpec((RC, 2), lambda i: (i, 0)),
            pl.BlockSpec((RC, 128), lambda i: (i, 0)),
            pl.BlockSpec((RC, 128), lambda i: (i, 0)),
            wspec, bspec, w64spec, bspec, w64spec, bspec,
        ],
        out_specs=pl.BlockSpec((RC, 128), lambda i: (i, 0)),
        out_shape=jax.ShapeDtypeStruct((M, 128), jnp.float32),
    )(pts, g0, g1, w1aT, b1a, w1bT, b1b, w1cT, b1c)


def _make_lmax(C):
    NC, NS = 2, 16
    NW = NC * NS
    M = B * N
    per_w = M // NW
    P = 8
    G = P * KNN
    nch = per_w // P
    mesh = plsc.VectorSubcoreMesh(core_axis_name="c", subcore_axis_name="s")

    @functools.partial(
        pl.kernel, mesh=mesh,
        out_type=jax.ShapeDtypeStruct((M, C), jnp.float32),
        scratch_types=[
            pltpu.VMEM((G,), jnp.int32),
            pltpu.VMEM((G,), jnp.int32),
            pltpu.VMEM((G, C), jnp.float32),
            pltpu.VMEM((G, C), jnp.float32),
            pltpu.VMEM((P, C), jnp.float32),
            pltpu.SemaphoreType.DMA,
            pltpu.SemaphoreType.DMA,
        ],
    )
    def kern(table, idxf, out, idx_v0, idx_v1, rows_v0, rows_v1, out_v,
             sem0, sem1):
        cid = lax.axis_index("c")
        sid = lax.axis_index("s")
        wid = sid * NC + cid
        base_pt = wid * per_w
        idxs = (idx_v0, idx_v1)
        rows = (rows_v0, rows_v1)
        sems = (sem0, sem1)

        def fire(g, buf):
            pt0 = base_pt + g * P
            pltpu.sync_copy(idxf.at[pl.ds(pt0 * KNN, G)], idxs[buf])
            pltpu.async_copy(table.at[idxs[buf]], rows[buf], sems[buf])

        def drain_compute(g, buf):
            pltpu.make_async_copy(table.at[idxs[buf]], rows[buf],
                                  sems[buf]).wait()
            rv = rows[buf]
            for p in range(P):
                for cc in range(C // 16):
                    sl = pl.ds(cc * 16, 16)
                    acc = rv[p * KNN, sl]
                    for r in range(1, KNN):
                        acc = jnp.maximum(acc, rv[p * KNN + r, sl])
                    out_v[p, sl] = acc
            pltpu.sync_copy(out_v, out.at[pl.ds(base_pt + g * P, P)])

        fire(0, 0)

        def pair(it, carry):
            g0 = 2 * it
            fire(g0 + 1, 1)
            drain_compute(g0, 0)

            @pl.when(it + 1 < nch // 2)
            def _():
                fire(g0 + 2, 0)

            drain_compute(g0 + 1, 1)
            return carry

        lax.fori_loop(0, nch // 2, pair, 0)

    return kern


def _make_gather():
    NC, NS = 2, 16
    NW = NC * NS
    M = B * N
    per_w = M // NW
    P = 256
    nch = per_w // P
    mesh = plsc.VectorSubcoreMesh(core_axis_name="c", subcore_axis_name="s")

    @functools.partial(
        pl.kernel, mesh=mesh,
        out_type=jax.ShapeDtypeStruct((M, 128), jnp.float32),
        scratch_types=[
            pltpu.VMEM((P,), jnp.int32),
            pltpu.VMEM((P, 128), jnp.float32),
            pltpu.SemaphoreType.DMA,
        ],
    )
    def kern(table, idxc, out, idx_v, rows_v, sem):
        cid = lax.axis_index("c")
        sid = lax.axis_index("s")
        wid = sid * NC + cid
        base = wid * per_w

        def chunk(g, carry):
            r0 = base + g * P
            pltpu.sync_copy(idxc.at[pl.ds(r0, P)], idx_v)
            pltpu.async_copy(table.at[idx_v], rows_v, sem).wait()
            pltpu.sync_copy(rows_v, out.at[pl.ds(r0, P)])
            return carry

        lax.fori_loop(0, nch, chunk, 0)

    return kern


@functools.cache
def _lmax(C):
    return _make_lmax(C)


@functools.cache
def _gather():
    return _make_gather()


def _lin_body(m_ref, wl_ref, bl_ref, wc_ref, bc_ref, out_ref):
    t = jnp.dot(m_ref[...], wl_ref[...],
                preferred_element_type=jnp.float32) + bl_ref[...]
    out_ref[...] = jnp.maximum(
        jnp.dot(t, wc_ref[...], preferred_element_type=jnp.float32)
        + bc_ref[...], 0.0)


def _lin(m1, wl1T, bl1, wc1T, bc1):
    P = 2048
    M = B * N
    return pl.pallas_call(
        _lin_body,
        grid=(M // P,),
        in_specs=[
            pl.BlockSpec((P, 128), lambda i: (i, 0)),
            pl.BlockSpec((128, 64), lambda i: (0, 0)),
            pl.BlockSpec((1, 64), lambda i: (0, 0)),
            pl.BlockSpec((64, 128), lambda i: (0, 0)),
            pl.BlockSpec((1, 128), lambda i: (0, 0)),
        ],
        out_specs=pl.BlockSpec((P, 128), lambda i: (i, 0)),
        out_shape=jax.ShapeDtypeStruct((M, 128), jnp.float32),
    )(m1, wl1T, bl1, wc1T, bc1)


def _tail_body(m2_ref, wl2_ref, bl2_ref, wc2_ref, bc2_ref,
               wma_ref, bma_ref, wmb_ref, bmb_ref, out_ref):
    t = jnp.dot(m2_ref[0], wl2_ref[...],
                preferred_element_type=jnp.float32) + bl2_ref[...]
    u = jnp.dot(t, wc2_ref[...],
                preferred_element_type=jnp.float32) + bc2_ref[...]
    mx = jnp.max(u, axis=0, keepdims=True)
    a = jnp.maximum(
        jnp.dot(mx, wma_ref[...], preferred_element_type=jnp.float32)
        + bma_ref[...], 0.0)
    out_ref[0] = jnp.dot(a, wmb_ref[...],
                         preferred_element_type=jnp.float32) + bmb_ref[...]


def _tail(m2, wl2T, bl2, wc2T, bc2, wmaT, bma, wmbT, bmb):
    full = lambda b: (0, 0)
    return pl.pallas_call(
        _tail_body,
        grid=(B,),
        in_specs=[
            pl.BlockSpec((1, N, 128), lambda b: (b, 0, 0)),
            pl.BlockSpec((128, 128), full),
            pl.BlockSpec((1, 128), full),
            pl.BlockSpec((128, 1024), full),
            pl.BlockSpec((1, 1024), full),
            pl.BlockSpec((1024, 512), full),
            pl.BlockSpec((1, 512), full),
            pl.BlockSpec((512, 512), full),
            pl.BlockSpec((1, 512), full),
        ],
        out_specs=pl.BlockSpec((1, 1, 512), lambda b: (b, 0, 0)),
        out_shape=jax.ShapeDtypeStruct((B, 1, 512), jnp.float32),
    )(m2, wl2T, bl2, wc2T, bc2, wmaT, bma, wmbT, bmb)


def kernel(pts, W1a, b1a, W1b, b1b, W1c, b1c, Wl1, bl1, Wc1, bc1,
           Wl2, bl2, Wc2, bc2, Wm2a, bm2a, Wm2b, bm2b):
    w1aT = jnp.pad(W1a, ((0, 0), (0, 2))).T
    ptsT = jnp.transpose(pts, (0, 2, 1))
    idx = _knn(pts, ptsT)
    ptsf = pts.reshape(B * N, 2)
    pts_pad = jnp.pad(ptsf, ((0, 0), (0, 126)))
    g0 = _gather()(pts_pad, idx[:, :, 0].reshape(-1))
    g1 = _gather()(pts_pad, idx[:, :, 1].reshape(-1))
    f1 = _cov_mlp1(ptsf, g0, g1, w1aT, b1a[None, :], W1b.T, b1b[None, :],
                   W1c.T, b1c[None, :])
    idxf = idx.reshape(-1)
    wl1T = jnp.pad(Wl1.T, ((0, 64), (0, 0)))
    m1 = _lmax(128)(f1, idxf)
    f2 = _lin(m1, wl1T, bl1[None, :], Wc1.T, bc1[None, :])
    m2 = _lmax(128)(f2, idxf)
    feat = _tail(m2.reshape(B, N, 128), Wl2.T, bl2[None, :], Wc2.T,
                 bc2[None, :], Wm2a.T, bm2a[None, :], Wm2b.T, bm2b[None, :])
    return feat


# speedup vs baseline: 4.9962x; 1.2358x over previous
import functools

import jax
import jax.numpy as jnp
from jax import lax
from jax.experimental import pallas as pl
from jax.experimental.pallas import tpu as pltpu
from jax.experimental.pallas import tpu_sc as plsc

B = 16
N = 2048
KNN = 16
R = 64
RC = 512


def _knn_body(xallT_ref, xblk_ref, idx_ref, work_ref):
    b = pl.program_id(0)
    xallT = xallT_ref[0]
    xblk = xblk_ref[0]
    xx_all = jnp.sum(xallT * xallT, axis=0, keepdims=True)
    xx_blk = jnp.sum(xblk * xblk, axis=1)
    inner = -2.0 * lax.dot_general(
        xblk, xallT, (((1,), (0,)), ((), ())),
        preferred_element_type=jnp.float32)
    work_ref[...] = (-xx_all - inner) - xx_blk[:, None]
    kiota = lax.broadcasted_iota(jnp.int32, (R, KNN), 1)
    neg = jnp.float32(-3e38)
    bigf = jnp.float32(N)

    def rowmax(v):
        w = jnp.maximum(v[:, :1024], v[:, 1024:])
        w = jnp.maximum(w[:, :512], w[:, 512:])
        w = jnp.maximum(w[:, :256], w[:, 256:512])
        m2 = jnp.max(w.reshape(R, 2, 128), axis=2)
        return jnp.max(m2, axis=1, keepdims=True)

    m0 = rowmax(work_ref[...])

    def step(carry):
        k, idx16, m = carry
        iotf = lax.broadcasted_iota(
            jnp.int32, (R, N), 1).astype(jnp.float32)
        cand = jnp.where(work_ref[...] >= m, iotf, bigf)
        c = jnp.minimum(cand[:, :1024], cand[:, 1024:])
        c = jnp.minimum(c[:, :512], c[:, 512:])
        c = jnp.minimum(c[:, :256], c[:, 256:512])
        a2 = jnp.min(c.reshape(R, 2, 128), axis=2)
        am = jnp.min(a2, axis=1, keepdims=True)
        idx16 = jnp.where(kiota == k, am.astype(jnp.int32), idx16)
        masked = jnp.where(iotf == am, neg, work_ref[...])
        work_ref[...] = masked
        return k + 1, idx16, rowmax(masked)

    _, idx16, _ = lax.while_loop(
        lambda c: c[0] < KNN, step,
        (jnp.int32(0), jnp.zeros((R, KNN), jnp.int32), m0))
    idx_ref[0] = idx16 + b * N


def _knn(pts, ptsT):
    nb = N // R
    return pl.pallas_call(
        _knn_body,
        grid=(B, nb),
        in_specs=[
            pl.BlockSpec((1, 2, N), lambda b, rb: (b, 0, 0)),
            pl.BlockSpec((1, R, 2), lambda b, rb: (b, rb, 0)),
        ],
        out_specs=pl.BlockSpec((1, R, KNN), lambda b, rb: (b, rb, 0)),
        out_shape=jax.ShapeDtypeStruct((B, N, KNN), jnp.int32),
        scratch_shapes=[
            pltpu.VMEM((R, N), jnp.float32),
        ],
        compiler_params=pltpu.CompilerParams(
            vmem_limit_bytes=100 * 1024 * 1024),
    )(ptsT, pts)


def _cov_mlp1_body(xblk_ref, g0_ref, g1_ref, w1a_ref, b1a_ref, w1b_ref,
                   b1b_ref, w1c_ref, b1c_ref, f1_ref):
    xblk = xblk_ref[...]
    g0 = g0_ref[:, 0:2]
    g1 = g1_ref[:, 0:2]
    cov = jnp.concatenate([
        (g0[:, 0] * g1[:, 0])[:, None],
        (g0[:, 0] * g1[:, 1])[:, None],
        (g0[:, 1] * g1[:, 0])[:, None],
        (g0[:, 1] * g1[:, 1])[:, None],
    ], axis=1)
    h8 = jnp.concatenate([xblk, cov, jnp.zeros((RC, 2), jnp.float32)],
                         axis=1)
    a = jnp.maximum(
        jnp.dot(h8, w1a_ref[...], preferred_element_type=jnp.float32)
        + b1a_ref[...], 0.0)
    a = jnp.maximum(
        jnp.dot(a, w1b_ref[...], preferred_element_type=jnp.float32)
        + b1b_ref[...], 0.0)
    a = jnp.maximum(
        jnp.dot(a, w1c_ref[...], preferred_element_type=jnp.float32)
        + b1c_ref[...], 0.0)
    f1_ref[...] = jnp.concatenate([a, jnp.zeros((RC, 64), jnp.float32)],
                                  axis=1)


def _cov_mlp1(pts, g0, g1, w1aT, b1a, w1bT, b1b, w1cT, b1c):
    M = B * N
    wspec = pl.BlockSpec((8, 64), lambda i: (0, 0))
    bspec = pl.BlockSpec((1, 64), lambda i: (0, 0))
    w64spec = pl.BlockSpec((64, 64), lambda i: (0, 0))
    return pl.pallas_call(
        _cov_mlp1_body,
        grid=(M // RC,),
        in_specs=[
            pl.BlockSpec((RC, 2), lambda i: (i, 0)),
            pl.BlockSpec((RC, 128), lambda i: (i, 0)),
            pl.BlockSpec((RC, 128), lambda i: (i, 0)),
            wspec, bspec, w64spec, bspec, w64spec, bspec,
        ],
        out_specs=pl.BlockSpec((RC, 128), lambda i: (i, 0)),
        out_shape=jax.ShapeDtypeStruct((M, 128), jnp.float32),
    )(pts, g0, g1, w1aT, b1a, w1bT, b1b, w1cT, b1c)


def _make_lmax(C):
    NC, NS = 2, 16
    NW = NC * NS
    M = B * N
    per_w = M // NW
    P = 8
    G = P * KNN
    nch = per_w // P
    mesh = plsc.VectorSubcoreMesh(core_axis_name="c", subcore_axis_name="s")

    @functools.partial(
        pl.kernel, mesh=mesh,
        out_type=jax.ShapeDtypeStruct((M, C), jnp.float32),
        scratch_types=[
            pltpu.VMEM((G,), jnp.int32),
            pltpu.VMEM((G,), jnp.int32),
            pltpu.VMEM((G, C), jnp.float32),
            pltpu.VMEM((G, C), jnp.float32),
            pltpu.VMEM((P, C), jnp.float32),
            pltpu.SemaphoreType.DMA,
            pltpu.SemaphoreType.DMA,
        ],
    )
    def kern(table, idxf, out, idx_v0, idx_v1, rows_v0, rows_v1, out_v,
             sem0, sem1):
        cid = lax.axis_index("c")
        sid = lax.axis_index("s")
        wid = sid * NC + cid
        base_pt = wid * per_w
        idxs = (idx_v0, idx_v1)
        rows = (rows_v0, rows_v1)
        sems = (sem0, sem1)

        def fire(g, buf):
            pt0 = base_pt + g * P
            pltpu.sync_copy(idxf.at[pl.ds(pt0 * KNN, G)], idxs[buf])
            pltpu.async_copy(table.at[idxs[buf]], rows[buf], sems[buf])

        def drain_compute(g, buf):
            pltpu.make_async_copy(table.at[idxs[buf]], rows[buf],
                                  sems[buf]).wait()
            rv = rows[buf]
            for p in range(P):
                for cc in range(C // 16):
                    sl = pl.ds(cc * 16, 16)
                    acc = rv[p * KNN, sl]
                    for r in range(1, KNN):
                        acc = jnp.maximum(acc, rv[p * KNN + r, sl])
                    out_v[p, sl] = acc
            pltpu.sync_copy(out_v, out.at[pl.ds(base_pt + g * P, P)])

        fire(0, 0)

        def pair(it, carry):
            g0 = 2 * it
            fire(g0 + 1, 1)
            drain_compute(g0, 0)

            @pl.when(it + 1 < nch // 2)
            def _():
                fire(g0 + 2, 0)

            drain_compute(g0 + 1, 1)
            return carry

        lax.fori_loop(0, nch // 2, pair, 0)

    return kern


def _make_gather():
    NC, NS = 2, 16
    NW = NC * NS
    M = B * N
    per_w = M // NW
    P = 256
    nch = per_w // P
    mesh = plsc.VectorSubcoreMesh(core_axis_name="c", subcore_axis_name="s")

    @functools.partial(
        pl.kernel, mesh=mesh,
        out_type=jax.ShapeDtypeStruct((M, 128), jnp.float32),
        scratch_types=[
            pltpu.VMEM((P,), jnp.int32),
            pltpu.VMEM((P, 128), jnp.float32),
            pltpu.SemaphoreType.DMA,
        ],
    )
    def kern(table, idxc, out, idx_v, rows_v, sem):
        cid = lax.axis_index("c")
        sid = lax.axis_index("s")
        wid = sid * NC + cid
        base = wid * per_w

        def chunk(g, carry):
            r0 = base + g * P
            pltpu.sync_copy(idxc.at[pl.ds(r0, P)], idx_v)
            pltpu.async_copy(table.at[idx_v], rows_v, sem).wait()
            pltpu.sync_copy(rows_v, out.at[pl.ds(r0, P)])
            return carry

        lax.fori_loop(0, nch, chunk, 0)

    return kern


@functools.cache
def _lmax(C):
    return _make_lmax(C)


@functools.cache
def _gather():
    return _make_gather()


def _lin_body(m_ref, wl_ref, bl_ref, wc_ref, bc_ref, out_ref):
    t = jnp.dot(m_ref[...], wl_ref[...],
                preferred_element_type=jnp.float32) + bl_ref[...]
    out_ref[...] = jnp.maximum(
        jnp.dot(t, wc_ref[...], preferred_element_type=jnp.float32)
        + bc_ref[...], 0.0)


def _lin(m1, wl1T, bl1, wc1T, bc1):
    P = 2048
    M = B * N
    return pl.pallas_call(
        _lin_body,
        grid=(M // P,),
        in_specs=[
            pl.BlockSpec((P, 128), lambda i: (i, 0)),
            pl.BlockSpec((128, 64), lambda i: (0, 0)),
            pl.BlockSpec((1, 64), lambda i: (0, 0)),
            pl.BlockSpec((64, 128), lambda i: (0, 0)),
            pl.BlockSpec((1, 128), lambda i: (0, 0)),
        ],
        out_specs=pl.BlockSpec((P, 128), lambda i: (i, 0)),
        out_shape=jax.ShapeDtypeStruct((M, 128), jnp.float32),
    )(m1, wl1T, bl1, wc1T, bc1)


def _tail_body(m2_ref, wl2_ref, bl2_ref, wc2_ref, bc2_ref,
               wma_ref, bma_ref, wmb_ref, bmb_ref, out_ref):
    t = jnp.dot(m2_ref[0], wl2_ref[...],
                preferred_element_type=jnp.float32) + bl2_ref[...]
    u = jnp.dot(t, wc2_ref[...],
                preferred_element_type=jnp.float32) + bc2_ref[...]
    mx = jnp.max(u, axis=0, keepdims=True)
    a = jnp.maximum(
        jnp.dot(mx, wma_ref[...], preferred_element_type=jnp.float32)
        + bma_ref[...], 0.0)
    out_ref[0] = jnp.dot(a, wmb_ref[...],
                         preferred_element_type=jnp.float32) + bmb_ref[...]


def _tail(m2, wl2T, bl2, wc2T, bc2, wmaT, bma, wmbT, bmb):
    full = lambda b: (0, 0)
    return pl.pallas_call(
        _tail_body,
        grid=(B,),
        in_specs=[
            pl.BlockSpec((1, N, 128), lambda b: (b, 0, 0)),
            pl.BlockSpec((128, 128), full),
            pl.BlockSpec((1, 128), full),
            pl.BlockSpec((128, 1024), full),
            pl.BlockSpec((1, 1024), full),
            pl.BlockSpec((1024, 512), full),
            pl.BlockSpec((1, 512), full),
            pl.BlockSpec((512, 512), full),
            pl.BlockSpec((1, 512), full),
        ],
        out_specs=pl.BlockSpec((1, 1, 512), lambda b: (b, 0, 0)),
        out_shape=jax.ShapeDtypeStruct((B, 1, 512), jnp.float32),
    )(m2, wl2T, bl2, wc2T, bc2, wmaT, bma, wmbT, bmb)


def kernel(pts, W1a, b1a, W1b, b1b, W1c, b1c, Wl1, bl1, Wc1, bc1,
           Wl2, bl2, Wc2, bc2, Wm2a, bm2a, Wm2b, bm2b):
    w1aT = jnp.pad(W1a, ((0, 0), (0, 2))).T
    ptsT = jnp.transpose(pts, (0, 2, 1))
    idx = _knn(pts, ptsT)
    ptsf = pts.reshape(B * N, 2)
    pts_pad = jnp.pad(ptsf, ((0, 0), (0, 126)))
    g0 = _gather()(pts_pad, idx[:, :, 0].reshape(-1))
    g1 = _gather()(pts_pad, idx[:, :, 1].reshape(-1))
    f1 = _cov_mlp1(ptsf, g0, g1, w1aT, b1a[None, :], W1b.T, b1b[None, :],
                   W1c.T, b1c[None, :])
    idxf = idx.reshape(-1)
    wl1T = jnp.pad(Wl1.T, ((0, 64), (0, 0)))
    m1 = _lmax(128)(f1, idxf)
    f2 = _lin(m1, wl1T, bl1[None, :], Wc1.T, bc1[None, :])
    m2 = _lmax(128)(f2, idxf)
    feat = _tail(m2.reshape(B, N, 128), Wl2.T, bl2[None, :], Wc2.T,
                 bc2[None, :], Wm2a.T, bm2a[None, :], Wm2b.T, bm2b[None, :])
    return feat


# speedup vs baseline: 7.6453x; 1.5302x over previous
import functools

import jax
import jax.numpy as jnp
from jax import lax
from jax.experimental import pallas as pl
from jax.experimental.pallas import tpu as pltpu
from jax.experimental.pallas import tpu_sc as plsc

B = 16
N = 2048
KNN = 16
R = 128
RC = 512


def _knn_body(xallT_ref, xblk_ref, idx_ref, work_ref):
    b = pl.program_id(0)
    xallT = xallT_ref[0]
    xblk = xblk_ref[0]
    xx_all = jnp.sum(xallT * xallT, axis=0, keepdims=True)
    xx_blk = jnp.sum(xblk * xblk, axis=1)
    inner = -2.0 * lax.dot_general(
        xblk, xallT, (((1,), (0,)), ((), ())),
        preferred_element_type=jnp.float32)
    work_ref[...] = (-xx_all - inner) - xx_blk[:, None]
    kiota = lax.broadcasted_iota(jnp.int32, (R, KNN), 1)
    neg = jnp.float32(-3e38)
    bigf = jnp.float32(N)

    def rowmax(v):
        w = jnp.maximum(v[:, :1024], v[:, 1024:])
        w = jnp.maximum(w[:, :512], w[:, 512:])
        w = jnp.maximum(w[:, :256], w[:, 256:512])
        m2 = jnp.max(w.reshape(R, 2, 128), axis=2)
        return jnp.max(m2, axis=1, keepdims=True)

    m0 = rowmax(work_ref[...])

    def step(carry):
        k, idx16, m = carry
        iotf = lax.broadcasted_iota(
            jnp.int32, (R, N), 1).astype(jnp.float32)
        cand = jnp.where(work_ref[...] >= m, iotf, bigf)
        c = jnp.minimum(cand[:, :1024], cand[:, 1024:])
        c = jnp.minimum(c[:, :512], c[:, 512:])
        c = jnp.minimum(c[:, :256], c[:, 256:512])
        a2 = jnp.min(c.reshape(R, 2, 128), axis=2)
        am = jnp.min(a2, axis=1, keepdims=True)
        idx16 = jnp.where(kiota == k, am.astype(jnp.int32), idx16)
        masked = jnp.where(iotf == am, neg, work_ref[...])
        work_ref[...] = masked
        return k + 1, idx16, rowmax(masked)

    _, idx16, _ = lax.while_loop(
        lambda c: c[0] < KNN, step,
        (jnp.int32(0), jnp.zeros((R, KNN), jnp.int32), m0))
    idx_ref[0] = idx16 + b * N


def _knn(pts, ptsT):
    nb = N // R
    return pl.pallas_call(
        _knn_body,
        grid=(B, nb),
        in_specs=[
            pl.BlockSpec((1, 2, N), lambda b, rb: (b, 0, 0)),
            pl.BlockSpec((1, R, 2), lambda b, rb: (b, rb, 0)),
        ],
        out_specs=pl.BlockSpec((1, R, KNN), lambda b, rb: (b, rb, 0)),
        out_shape=jax.ShapeDtypeStruct((B, N, KNN), jnp.int32),
        scratch_shapes=[
            pltpu.VMEM((R, N), jnp.float32),
        ],
        compiler_params=pltpu.CompilerParams(
            vmem_limit_bytes=127 * 1024 * 1024),
    )(ptsT, pts)


def _cov_mlp1_body(xblk_ref, g0_ref, g1_ref, w1a_ref, b1a_ref, w1b_ref,
                   b1b_ref, w1c_ref, b1c_ref, f1_ref):
    xblk = xblk_ref[...]
    g0 = g0_ref[:, 0:2]
    g1 = g1_ref[:, 0:2]
    cov = jnp.concatenate([
        (g0[:, 0] * g1[:, 0])[:, None],
        (g0[:, 0] * g1[:, 1])[:, None],
        (g0[:, 1] * g1[:, 0])[:, None],
        (g0[:, 1] * g1[:, 1])[:, None],
    ], axis=1)
    h8 = jnp.concatenate([xblk, cov, jnp.zeros((RC, 2), jnp.float32)],
                         axis=1)
    a = jnp.maximum(
        jnp.dot(h8, w1a_ref[...], preferred_element_type=jnp.float32)
        + b1a_ref[...], 0.0)
    a = jnp.maximum(
        jnp.dot(a, w1b_ref[...], preferred_element_type=jnp.float32)
        + b1b_ref[...], 0.0)
    a = jnp.maximum(
        jnp.dot(a, w1c_ref[...], preferred_element_type=jnp.float32)
        + b1c_ref[...], 0.0)
    f1_ref[...] = jnp.concatenate([a, jnp.zeros((RC, 64), jnp.float32)],
                                  axis=1)


def _cov_mlp1(pts, g0, g1, w1aT, b1a, w1bT, b1b, w1cT, b1c):
    M = B * N
    wspec = pl.BlockSpec((8, 64), lambda i: (0, 0))
    bspec = pl.BlockSpec((1, 64), lambda i: (0, 0))
    w64spec = pl.BlockSpec((64, 64), lambda i: (0, 0))
    return pl.pallas_call(
        _cov_mlp1_body,
        grid=(M // RC,),
        in_specs=[
            pl.BlockSpec((RC, 2), lambda i: (i, 0)),
            pl.BlockSpec((RC, 128), lambda i: (i, 0)),
            pl.BlockSpec((RC, 128), lambda i: (i, 0)),
            wspec, bspec, w64spec, bspec, w64spec, bspec,
        ],
        out_specs=pl.BlockSpec((RC, 128), lambda i: (i, 0)),
        out_shape=jax.ShapeDtypeStruct((M, 128), jnp.float32),
    )(pts, g0, g1, w1aT, b1a, w1bT, b1b, w1cT, b1c)


def _make_lmax(C):
    NC, NS = 2, 16
    NW = NC * NS
    M = B * N
    per_w = M // NW
    P = 8
    G = P * KNN
    nch = per_w // P
    mesh = plsc.VectorSubcoreMesh(core_axis_name="c", subcore_axis_name="s")

    @functools.partial(
        pl.kernel, mesh=mesh,
        out_type=jax.ShapeDtypeStruct((M, C), jnp.float32),
        scratch_types=[
            pltpu.VMEM((G,), jnp.int32),
            pltpu.VMEM((G,), jnp.int32),
            pltpu.VMEM((G, C), jnp.float32),
            pltpu.VMEM((G, C), jnp.float32),
            pltpu.VMEM((P, C), jnp.float32),
            pltpu.SemaphoreType.DMA,
            pltpu.SemaphoreType.DMA,
        ],
    )
    def kern(table, idxf, out, idx_v0, idx_v1, rows_v0, rows_v1, out_v,
             sem0, sem1):
        cid = lax.axis_index("c")
        sid = lax.axis_index("s")
        wid = sid * NC + cid
        base_pt = wid * per_w
        idxs = (idx_v0, idx_v1)
        rows = (rows_v0, rows_v1)
        sems = (sem0, sem1)

        def fire(g, buf):
            pt0 = base_pt + g * P
            pltpu.sync_copy(idxf.at[pl.ds(pt0 * KNN, G)], idxs[buf])
            pltpu.async_copy(table.at[idxs[buf]], rows[buf], sems[buf])

        def drain_compute(g, buf):
            pltpu.make_async_copy(table.at[idxs[buf]], rows[buf],
                                  sems[buf]).wait()
            rv = rows[buf]
            for p in range(P):
                for cc in range(C // 16):
                    sl = pl.ds(cc * 16, 16)
                    acc = rv[p * KNN, sl]
                    for r in range(1, KNN):
                        acc = jnp.maximum(acc, rv[p * KNN + r, sl])
                    out_v[p, sl] = acc
            pltpu.sync_copy(out_v, out.at[pl.ds(base_pt + g * P, P)])

        fire(0, 0)

        def pair(it, carry):
            g0 = 2 * it
            fire(g0 + 1, 1)
            drain_compute(g0, 0)

            @pl.when(it + 1 < nch // 2)
            def _():
                fire(g0 + 2, 0)

            drain_compute(g0 + 1, 1)
            return carry

        lax.fori_loop(0, nch // 2, pair, 0)

    return kern


def _make_gather():
    NC, NS = 2, 16
    NW = NC * NS
    M = B * N
    per_w = M // NW
    P = 256
    nch = per_w // P
    mesh = plsc.VectorSubcoreMesh(core_axis_name="c", subcore_axis_name="s")

    @functools.partial(
        pl.kernel, mesh=mesh,
        out_type=jax.ShapeDtypeStruct((M, 128), jnp.float32),
        scratch_types=[
            pltpu.VMEM((P,), jnp.int32),
            pltpu.VMEM((P, 128), jnp.float32),
            pltpu.SemaphoreType.DMA,
        ],
    )
    def kern(table, idxc, out, idx_v, rows_v, sem):
        cid = lax.axis_index("c")
        sid = lax.axis_index("s")
        wid = sid * NC + cid
        base = wid * per_w

        def chunk(g, carry):
            r0 = base + g * P
            pltpu.sync_copy(idxc.at[pl.ds(r0, P)], idx_v)
            pltpu.async_copy(table.at[idx_v], rows_v, sem).wait()
            pltpu.sync_copy(rows_v, out.at[pl.ds(r0, P)])
            return carry

        lax.fori_loop(0, nch, chunk, 0)

    return kern


@functools.cache
def _lmax(C):
    return _make_lmax(C)


@functools.cache
def _gather():
    return _make_gather()


def _lin_body(m_ref, wl_ref, bl_ref, wc_ref, bc_ref, out_ref):
    t = jnp.dot(m_ref[...], wl_ref[...],
                preferred_element_type=jnp.float32) + bl_ref[...]
    out_ref[...] = jnp.maximum(
        jnp.dot(t, wc_ref[...], preferred_element_type=jnp.float32)
        + bc_ref[...], 0.0)


def _lin(m1, wl1T, bl1, wc1T, bc1):
    P = 2048
    M = B * N
    return pl.pallas_call(
        _lin_body,
        grid=(M // P,),
        in_specs=[
            pl.BlockSpec((P, 128), lambda i: (i, 0)),
            pl.BlockSpec((128, 64), lambda i: (0, 0)),
            pl.BlockSpec((1, 64), lambda i: (0, 0)),
            pl.BlockSpec((64, 128), lambda i: (0, 0)),
            pl.BlockSpec((1, 128), lambda i: (0, 0)),
        ],
        out_specs=pl.BlockSpec((P, 128), lambda i: (i, 0)),
        out_shape=jax.ShapeDtypeStruct((M, 128), jnp.float32),
    )(m1, wl1T, bl1, wc1T, bc1)


def _tail_body(m2_ref, wl2_ref, bl2_ref, wc2_ref, bc2_ref,
               wma_ref, bma_ref, wmb_ref, bmb_ref, out_ref):
    t = jnp.dot(m2_ref[0], wl2_ref[...],
                preferred_element_type=jnp.float32) + bl2_ref[...]
    u = jnp.dot(t, wc2_ref[...],
                preferred_element_type=jnp.float32) + bc2_ref[...]
    mx = jnp.max(u, axis=0, keepdims=True)
    a = jnp.maximum(
        jnp.dot(mx, wma_ref[...], preferred_element_type=jnp.float32)
        + bma_ref[...], 0.0)
    out_ref[0] = jnp.dot(a, wmb_ref[...],
                         preferred_element_type=jnp.float32) + bmb_ref[...]


def _tail(m2, wl2T, bl2, wc2T, bc2, wmaT, bma, wmbT, bmb):
    full = lambda b: (0, 0)
    return pl.pallas_call(
        _tail_body,
        grid=(B,),
        in_specs=[
            pl.BlockSpec((1, N, 128), lambda b: (b, 0, 0)),
            pl.BlockSpec((128, 128), full),
            pl.BlockSpec((1, 128), full),
            pl.BlockSpec((128, 1024), full),
            pl.BlockSpec((1, 1024), full),
            pl.BlockSpec((1024, 512), full),
            pl.BlockSpec((1, 512), full),
            pl.BlockSpec((512, 512), full),
            pl.BlockSpec((1, 512), full),
        ],
        out_specs=pl.BlockSpec((1, 1, 512), lambda b: (b, 0, 0)),
        out_shape=jax.ShapeDtypeStruct((B, 1, 512), jnp.float32),
    )(m2, wl2T, bl2, wc2T, bc2, wmaT, bma, wmbT, bmb)


def kernel(pts, W1a, b1a, W1b, b1b, W1c, b1c, Wl1, bl1, Wc1, bc1,
           Wl2, bl2, Wc2, bc2, Wm2a, bm2a, Wm2b, bm2b):
    w1aT = jnp.pad(W1a, ((0, 0), (0, 2))).T
    ptsT = jnp.transpose(pts, (0, 2, 1))
    idx = _knn(pts, ptsT)
    ptsf = pts.reshape(B * N, 2)
    pts_pad = jnp.pad(ptsf, ((0, 0), (0, 126)))
    g0 = _gather()(pts_pad, idx[:, :, 0].reshape(-1))
    g1 = _gather()(pts_pad, idx[:, :, 1].reshape(-1))
    f1 = _cov_mlp1(ptsf, g0, g1, w1aT, b1a[None, :], W1b.T, b1b[None, :],
                   W1c.T, b1c[None, :])
    idxf = idx.reshape(-1)
    wl1T = jnp.pad(Wl1.T, ((0, 64), (0, 0)))
    m1 = _lmax(128)(f1, idxf)
    f2 = _lin(m1, wl1T, bl1[None, :], Wc1.T, bc1[None, :])
    m2 = _lmax(128)(f2, idxf)
    feat = _tail(m2.reshape(B, N, 128), Wl2.T, bl2[None, :], Wc2.T,
                 bc2[None, :], Wm2a.T, bm2a[None, :], Wm2b.T, bm2b[None, :])
    return feat


# speedup vs baseline: 8.6774x; 1.1350x over previous
import functools

import jax
import jax.numpy as jnp
from jax import lax
from jax.experimental import pallas as pl
from jax.experimental.pallas import tpu as pltpu
from jax.experimental.pallas import tpu_sc as plsc

B = 16
N = 2048
KNN = 16
R = 256
RC = 512


def _knn_body(xallT_ref, xblk_ref, idx_ref, work_ref):
    b = pl.program_id(0)
    xallT = xallT_ref[0]
    xblk = xblk_ref[0]
    xx_all = jnp.sum(xallT * xallT, axis=0, keepdims=True)
    xx_blk = jnp.sum(xblk * xblk, axis=1)
    inner = -2.0 * lax.dot_general(
        xblk, xallT, (((1,), (0,)), ((), ())),
        preferred_element_type=jnp.float32)
    work_ref[...] = (-xx_all - inner) - xx_blk[:, None]
    kiota = lax.broadcasted_iota(jnp.int32, (R, KNN), 1)
    neg = jnp.float32(-3e38)
    bigf = jnp.float32(N)

    def rowmax(v):
        w = jnp.maximum(v[:, :1024], v[:, 1024:])
        w = jnp.maximum(w[:, :512], w[:, 512:])
        w = jnp.maximum(w[:, :256], w[:, 256:512])
        m2 = jnp.max(w.reshape(R, 2, 128), axis=2)
        return jnp.max(m2, axis=1, keepdims=True)

    m0 = rowmax(work_ref[...])

    def step(carry):
        k, idx16, m = carry
        iotf = lax.broadcasted_iota(
            jnp.int32, (R, N), 1).astype(jnp.float32)
        cand = jnp.where(work_ref[...] >= m, iotf, bigf)
        c = jnp.minimum(cand[:, :1024], cand[:, 1024:])
        c = jnp.minimum(c[:, :512], c[:, 512:])
        c = jnp.minimum(c[:, :256], c[:, 256:512])
        a2 = jnp.min(c.reshape(R, 2, 128), axis=2)
        am = jnp.min(a2, axis=1, keepdims=True)
        idx16 = jnp.where(kiota == k, am.astype(jnp.int32), idx16)
        masked = jnp.where(iotf == am, neg, work_ref[...])
        work_ref[...] = masked
        return k + 1, idx16, rowmax(masked)

    _, idx16, _ = lax.while_loop(
        lambda c: c[0] < KNN, step,
        (jnp.int32(0), jnp.zeros((R, KNN), jnp.int32), m0))
    idx_ref[0] = idx16 + b * N


def _knn(pts, ptsT):
    nb = N // R
    return pl.pallas_call(
        _knn_body,
        grid=(B, nb),
        in_specs=[
            pl.BlockSpec((1, 2, N), lambda b, rb: (b, 0, 0)),
            pl.BlockSpec((1, R, 2), lambda b, rb: (b, rb, 0)),
        ],
        out_specs=pl.BlockSpec((1, R, KNN), lambda b, rb: (b, rb, 0)),
        out_shape=jax.ShapeDtypeStruct((B, N, KNN), jnp.int32),
        scratch_shapes=[
            pltpu.VMEM((R, N), jnp.float32),
        ],
        compiler_params=pltpu.CompilerParams(
            vmem_limit_bytes=127 * 1024 * 1024),
    )(ptsT, pts)


def _cov_mlp1_body(xblk_ref, g0_ref, g1_ref, w1a_ref, b1a_ref, w1b_ref,
                   b1b_ref, w1c_ref, b1c_ref, f1_ref):
    xblk = xblk_ref[...]
    g0 = g0_ref[:, 0:2]
    g1 = g1_ref[:, 0:2]
    cov = jnp.concatenate([
        (g0[:, 0] * g1[:, 0])[:, None],
        (g0[:, 0] * g1[:, 1])[:, None],
        (g0[:, 1] * g1[:, 0])[:, None],
        (g0[:, 1] * g1[:, 1])[:, None],
    ], axis=1)
    h8 = jnp.concatenate([xblk, cov, jnp.zeros((RC, 2), jnp.float32)],
                         axis=1)
    a = jnp.maximum(
        jnp.dot(h8, w1a_ref[...], preferred_element_type=jnp.float32)
        + b1a_ref[...], 0.0)
    a = jnp.maximum(
        jnp.dot(a, w1b_ref[...], preferred_element_type=jnp.float32)
        + b1b_ref[...], 0.0)
    a = jnp.maximum(
        jnp.dot(a, w1c_ref[...], preferred_element_type=jnp.float32)
        + b1c_ref[...], 0.0)
    f1_ref[...] = jnp.concatenate([a, jnp.zeros((RC, 64), jnp.float32)],
                                  axis=1)


def _cov_mlp1(pts, g0, g1, w1aT, b1a, w1bT, b1b, w1cT, b1c):
    M = B * N
    wspec = pl.BlockSpec((8, 64), lambda i: (0, 0))
    bspec = pl.BlockSpec((1, 64), lambda i: (0, 0))
    w64spec = pl.BlockSpec((64, 64), lambda i: (0, 0))
    return pl.pallas_call(
        _cov_mlp1_body,
        grid=(M // RC,),
        in_specs=[
            pl.BlockSpec((RC, 2), lambda i: (i, 0)),
            pl.BlockSpec((RC, 128), lambda i: (i, 0)),
            pl.BlockSpec((RC, 128), lambda i: (i, 0)),
            wspec, bspec, w64spec, bspec, w64spec, bspec,
        ],
        out_specs=pl.BlockSpec((RC, 128), lambda i: (i, 0)),
        out_shape=jax.ShapeDtypeStruct((M, 128), jnp.float32),
    )(pts, g0, g1, w1aT, b1a, w1bT, b1b, w1cT, b1c)


def _make_lmax(C):
    NC, NS = 2, 16
    NW = NC * NS
    M = B * N
    per_w = M // NW
    P = 8
    G = P * KNN
    nch = per_w // P
    mesh = plsc.VectorSubcoreMesh(core_axis_name="c", subcore_axis_name="s")

    @functools.partial(
        pl.kernel, mesh=mesh,
        out_type=jax.ShapeDtypeStruct((M, C), jnp.float32),
        scratch_types=[
            pltpu.VMEM((G,), jnp.int32),
            pltpu.VMEM((G,), jnp.int32),
            pltpu.VMEM((G, C), jnp.float32),
            pltpu.VMEM((G, C), jnp.float32),
            pltpu.VMEM((P, C), jnp.float32),
            pltpu.SemaphoreType.DMA,
            pltpu.SemaphoreType.DMA,
        ],
    )
    def kern(table, idxf, out, idx_v0, idx_v1, rows_v0, rows_v1, out_v,
             sem0, sem1):
        cid = lax.axis_index("c")
        sid = lax.axis_index("s")
        wid = sid * NC + cid
        base_pt = wid * per_w
        idxs = (idx_v0, idx_v1)
        rows = (rows_v0, rows_v1)
        sems = (sem0, sem1)

        def fire(g, buf):
            pt0 = base_pt + g * P
            pltpu.sync_copy(idxf.at[pl.ds(pt0 * KNN, G)], idxs[buf])
            pltpu.async_copy(table.at[idxs[buf]], rows[buf], sems[buf])

        def drain_compute(g, buf):
            pltpu.make_async_copy(table.at[idxs[buf]], rows[buf],
                                  sems[buf]).wait()
            rv = rows[buf]
            for p in range(P):
                for cc in range(C // 16):
                    sl = pl.ds(cc * 16, 16)
                    acc = rv[p * KNN, sl]
                    for r in range(1, KNN):
                        acc = jnp.maximum(acc, rv[p * KNN + r, sl])
                    out_v[p, sl] = acc
            pltpu.sync_copy(out_v, out.at[pl.ds(base_pt + g * P, P)])

        fire(0, 0)

        def pair(it, carry):
            g0 = 2 * it
            fire(g0 + 1, 1)
            drain_compute(g0, 0)

            @pl.when(it + 1 < nch // 2)
            def _():
                fire(g0 + 2, 0)

            drain_compute(g0 + 1, 1)
            return carry

        lax.fori_loop(0, nch // 2, pair, 0)

    return kern


def _make_gather():
    NC, NS = 2, 16
    NW = NC * NS
    M = B * N
    per_w = M // NW
    P = 256
    nch = per_w // P
    mesh = plsc.VectorSubcoreMesh(core_axis_name="c", subcore_axis_name="s")

    @functools.partial(
        pl.kernel, mesh=mesh,
        out_type=jax.ShapeDtypeStruct((M, 128), jnp.float32),
        scratch_types=[
            pltpu.VMEM((P,), jnp.int32),
            pltpu.VMEM((P, 128), jnp.float32),
            pltpu.SemaphoreType.DMA,
        ],
    )
    def kern(table, idxc, out, idx_v, rows_v, sem):
        cid = lax.axis_index("c")
        sid = lax.axis_index("s")
        wid = sid * NC + cid
        base = wid * per_w

        def chunk(g, carry):
            r0 = base + g * P
            pltpu.sync_copy(idxc.at[pl.ds(r0, P)], idx_v)
            pltpu.async_copy(table.at[idx_v], rows_v, sem).wait()
            pltpu.sync_copy(rows_v, out.at[pl.ds(r0, P)])
            return carry

        lax.fori_loop(0, nch, chunk, 0)

    return kern


@functools.cache
def _lmax(C):
    return _make_lmax(C)


@functools.cache
def _gather():
    return _make_gather()


def _lin_body(m_ref, wl_ref, bl_ref, wc_ref, bc_ref, out_ref):
    t = jnp.dot(m_ref[...], wl_ref[...],
                preferred_element_type=jnp.float32) + bl_ref[...]
    out_ref[...] = jnp.maximum(
        jnp.dot(t, wc_ref[...], preferred_element_type=jnp.float32)
        + bc_ref[...], 0.0)


def _lin(m1, wl1T, bl1, wc1T, bc1):
    P = 2048
    M = B * N
    return pl.pallas_call(
        _lin_body,
        grid=(M // P,),
        in_specs=[
            pl.BlockSpec((P, 128), lambda i: (i, 0)),
            pl.BlockSpec((128, 64), lambda i: (0, 0)),
            pl.BlockSpec((1, 64), lambda i: (0, 0)),
            pl.BlockSpec((64, 128), lambda i: (0, 0)),
            pl.BlockSpec((1, 128), lambda i: (0, 0)),
        ],
        out_specs=pl.BlockSpec((P, 128), lambda i: (i, 0)),
        out_shape=jax.ShapeDtypeStruct((M, 128), jnp.float32),
    )(m1, wl1T, bl1, wc1T, bc1)


def _tail_body(m2_ref, wl2_ref, bl2_ref, wc2_ref, bc2_ref,
               wma_ref, bma_ref, wmb_ref, bmb_ref, out_ref):
    t = jnp.dot(m2_ref[0], wl2_ref[...],
                preferred_element_type=jnp.float32) + bl2_ref[...]
    u = jnp.dot(t, wc2_ref[...],
                preferred_element_type=jnp.float32) + bc2_ref[...]
    mx = jnp.max(u, axis=0, keepdims=True)
    a = jnp.maximum(
        jnp.dot(mx, wma_ref[...], preferred_element_type=jnp.float32)
        + bma_ref[...], 0.0)
    out_ref[0] = jnp.dot(a, wmb_ref[...],
                         preferred_element_type=jnp.float32) + bmb_ref[...]


def _tail(m2, wl2T, bl2, wc2T, bc2, wmaT, bma, wmbT, bmb):
    full = lambda b: (0, 0)
    return pl.pallas_call(
        _tail_body,
        grid=(B,),
        in_specs=[
            pl.BlockSpec((1, N, 128), lambda b: (b, 0, 0)),
            pl.BlockSpec((128, 128), full),
            pl.BlockSpec((1, 128), full),
            pl.BlockSpec((128, 1024), full),
            pl.BlockSpec((1, 1024), full),
            pl.BlockSpec((1024, 512), full),
            pl.BlockSpec((1, 512), full),
            pl.BlockSpec((512, 512), full),
            pl.BlockSpec((1, 512), full),
        ],
        out_specs=pl.BlockSpec((1, 1, 512), lambda b: (b, 0, 0)),
        out_shape=jax.ShapeDtypeStruct((B, 1, 512), jnp.float32),
    )(m2, wl2T, bl2, wc2T, bc2, wmaT, bma, wmbT, bmb)


def kernel(pts, W1a, b1a, W1b, b1b, W1c, b1c, Wl1, bl1, Wc1, bc1,
           Wl2, bl2, Wc2, bc2, Wm2a, bm2a, Wm2b, bm2b):
    w1aT = jnp.pad(W1a, ((0, 0), (0, 2))).T
    ptsT = jnp.transpose(pts, (0, 2, 1))
    idx = _knn(pts, ptsT)
    ptsf = pts.reshape(B * N, 2)
    pts_pad = jnp.pad(ptsf, ((0, 0), (0, 126)))
    g0 = _gather()(pts_pad, idx[:, :, 0].reshape(-1))
    g1 = _gather()(pts_pad, idx[:, :, 1].reshape(-1))
    f1 = _cov_mlp1(ptsf, g0, g1, w1aT, b1a[None, :], W1b.T, b1b[None, :],
                   W1c.T, b1c[None, :])
    idxf = idx.reshape(-1)
    wl1T = jnp.pad(Wl1.T, ((0, 64), (0, 0)))
    m1 = _lmax(128)(f1, idxf)
    f2 = _lin(m1, wl1T, bl1[None, :], Wc1.T, bc1[None, :])
    m2 = _lmax(128)(f2, idxf)
    feat = _tail(m2.reshape(B, N, 128), Wl2.T, bl2[None, :], Wc2.T,
                 bc2[None, :], Wm2a.T, bm2a[None, :], Wm2b.T, bm2b[None, :])
    return feat


# speedup vs baseline: 9.2449x; 1.0654x over previous
import functools

import jax
import jax.numpy as jnp
from jax import lax
from jax.experimental import pallas as pl
from jax.experimental.pallas import tpu as pltpu
from jax.experimental.pallas import tpu_sc as plsc

B = 16
N = 2048
KNN = 16
R = 512
RC = 512


def _knn_body(xallT_ref, xblk_ref, idx_ref, work_ref):
    b = pl.program_id(0)
    xallT = xallT_ref[0]
    xblk = xblk_ref[0]
    xx_all = jnp.sum(xallT * xallT, axis=0, keepdims=True)
    xx_blk = jnp.sum(xblk * xblk, axis=1)
    inner = -2.0 * lax.dot_general(
        xblk, xallT, (((1,), (0,)), ((), ())),
        preferred_element_type=jnp.float32)
    work_ref[...] = (-xx_all - inner) - xx_blk[:, None]
    kiota = lax.broadcasted_iota(jnp.int32, (R, KNN), 1)
    neg = jnp.float32(-3e38)
    bigf = jnp.float32(N)

    def rowmax(v):
        w = jnp.maximum(v[:, :1024], v[:, 1024:])
        w = jnp.maximum(w[:, :512], w[:, 512:])
        w = jnp.maximum(w[:, :256], w[:, 256:512])
        m2 = jnp.max(w.reshape(R, 2, 128), axis=2)
        return jnp.max(m2, axis=1, keepdims=True)

    m0 = rowmax(work_ref[...])

    def step(carry):
        k, idx16, m = carry
        iotf = lax.broadcasted_iota(
            jnp.int32, (R, N), 1).astype(jnp.float32)
        cand = jnp.where(work_ref[...] >= m, iotf, bigf)
        c = jnp.minimum(cand[:, :1024], cand[:, 1024:])
        c = jnp.minimum(c[:, :512], c[:, 512:])
        c = jnp.minimum(c[:, :256], c[:, 256:512])
        a2 = jnp.min(c.reshape(R, 2, 128), axis=2)
        am = jnp.min(a2, axis=1, keepdims=True)
        idx16 = jnp.where(kiota == k, am.astype(jnp.int32), idx16)
        masked = jnp.where(iotf == am, neg, work_ref[...])
        work_ref[...] = masked
        return k + 1, idx16, rowmax(masked)

    _, idx16, _ = lax.while_loop(
        lambda c: c[0] < KNN, step,
        (jnp.int32(0), jnp.zeros((R, KNN), jnp.int32), m0))
    idx_ref[0] = idx16 + b * N


def _knn(pts, ptsT):
    nb = N // R
    return pl.pallas_call(
        _knn_body,
        grid=(B, nb),
        in_specs=[
            pl.BlockSpec((1, 2, N), lambda b, rb: (b, 0, 0)),
            pl.BlockSpec((1, R, 2), lambda b, rb: (b, rb, 0)),
        ],
        out_specs=pl.BlockSpec((1, R, KNN), lambda b, rb: (b, rb, 0)),
        out_shape=jax.ShapeDtypeStruct((B, N, KNN), jnp.int32),
        scratch_shapes=[
            pltpu.VMEM((R, N), jnp.float32),
        ],
        compiler_params=pltpu.CompilerParams(
            vmem_limit_bytes=127 * 1024 * 1024),
    )(ptsT, pts)


def _cov_mlp1_body(xblk_ref, g0_ref, g1_ref, w1a_ref, b1a_ref, w1b_ref,
                   b1b_ref, w1c_ref, b1c_ref, f1_ref):
    xblk = xblk_ref[...]
    g0 = g0_ref[:, 0:2]
    g1 = g1_ref[:, 0:2]
    cov = jnp.concatenate([
        (g0[:, 0] * g1[:, 0])[:, None],
        (g0[:, 0] * g1[:, 1])[:, None],
        (g0[:, 1] * g1[:, 0])[:, None],
        (g0[:, 1] * g1[:, 1])[:, None],
    ], axis=1)
    h8 = jnp.concatenate([xblk, cov, jnp.zeros((RC, 2), jnp.float32)],
                         axis=1)
    a = jnp.maximum(
        jnp.dot(h8, w1a_ref[...], preferred_element_type=jnp.float32)
        + b1a_ref[...], 0.0)
    a = jnp.maximum(
        jnp.dot(a, w1b_ref[...], preferred_element_type=jnp.float32)
        + b1b_ref[...], 0.0)
    a = jnp.maximum(
        jnp.dot(a, w1c_ref[...], preferred_element_type=jnp.float32)
        + b1c_ref[...], 0.0)
    f1_ref[...] = jnp.concatenate([a, jnp.zeros((RC, 64), jnp.float32)],
                                  axis=1)


def _cov_mlp1(pts, g0, g1, w1aT, b1a, w1bT, b1b, w1cT, b1c):
    M = B * N
    wspec = pl.BlockSpec((8, 64), lambda i: (0, 0))
    bspec = pl.BlockSpec((1, 64), lambda i: (0, 0))
    w64spec = pl.BlockSpec((64, 64), lambda i: (0, 0))
    return pl.pallas_call(
        _cov_mlp1_body,
        grid=(M // RC,),
        in_specs=[
            pl.BlockSpec((RC, 2), lambda i: (i, 0)),
            pl.BlockSpec((RC, 128), lambda i: (i, 0)),
            pl.BlockSpec((RC, 128), lambda i: (i, 0)),
            wspec, bspec, w64spec, bspec, w64spec, bspec,
        ],
        out_specs=pl.BlockSpec((RC, 128), lambda i: (i, 0)),
        out_shape=jax.ShapeDtypeStruct((M, 128), jnp.float32),
    )(pts, g0, g1, w1aT, b1a, w1bT, b1b, w1cT, b1c)


def _make_lmax(C):
    NC, NS = 2, 16
    NW = NC * NS
    M = B * N
    per_w = M // NW
    P = 8
    G = P * KNN
    nch = per_w // P
    mesh = plsc.VectorSubcoreMesh(core_axis_name="c", subcore_axis_name="s")

    @functools.partial(
        pl.kernel, mesh=mesh,
        out_type=jax.ShapeDtypeStruct((M, C), jnp.float32),
        scratch_types=[
            pltpu.VMEM((G,), jnp.int32),
            pltpu.VMEM((G,), jnp.int32),
            pltpu.VMEM((G, C), jnp.float32),
            pltpu.VMEM((G, C), jnp.float32),
            pltpu.VMEM((P, C), jnp.float32),
            pltpu.SemaphoreType.DMA,
            pltpu.SemaphoreType.DMA,
        ],
    )
    def kern(table, idxf, out, idx_v0, idx_v1, rows_v0, rows_v1, out_v,
             sem0, sem1):
        cid = lax.axis_index("c")
        sid = lax.axis_index("s")
        wid = sid * NC + cid
        base_pt = wid * per_w
        idxs = (idx_v0, idx_v1)
        rows = (rows_v0, rows_v1)
        sems = (sem0, sem1)

        def fire(g, buf):
            pt0 = base_pt + g * P
            pltpu.sync_copy(idxf.at[pl.ds(pt0 * KNN, G)], idxs[buf])
            pltpu.async_copy(table.at[idxs[buf]], rows[buf], sems[buf])

        def drain_compute(g, buf):
            pltpu.make_async_copy(table.at[idxs[buf]], rows[buf],
                                  sems[buf]).wait()
            rv = rows[buf]
            for p in range(P):
                for cc in range(C // 16):
                    sl = pl.ds(cc * 16, 16)
                    acc = rv[p * KNN, sl]
                    for r in range(1, KNN):
                        acc = jnp.maximum(acc, rv[p * KNN + r, sl])
                    out_v[p, sl] = acc
            pltpu.sync_copy(out_v, out.at[pl.ds(base_pt + g * P, P)])

        fire(0, 0)

        def pair(it, carry):
            g0 = 2 * it
            fire(g0 + 1, 1)
            drain_compute(g0, 0)

            @pl.when(it + 1 < nch // 2)
            def _():
                fire(g0 + 2, 0)

            drain_compute(g0 + 1, 1)
            return carry

        lax.fori_loop(0, nch // 2, pair, 0)

    return kern


def _make_gather():
    NC, NS = 2, 16
    NW = NC * NS
    M = B * N
    per_w = M // NW
    P = 256
    nch = per_w // P
    mesh = plsc.VectorSubcoreMesh(core_axis_name="c", subcore_axis_name="s")

    @functools.partial(
        pl.kernel, mesh=mesh,
        out_type=jax.ShapeDtypeStruct((M, 128), jnp.float32),
        scratch_types=[
            pltpu.VMEM((P,), jnp.int32),
            pltpu.VMEM((P, 128), jnp.float32),
            pltpu.SemaphoreType.DMA,
        ],
    )
    def kern(table, idxc, out, idx_v, rows_v, sem):
        cid = lax.axis_index("c")
        sid = lax.axis_index("s")
        wid = sid * NC + cid
        base = wid * per_w

        def chunk(g, carry):
            r0 = base + g * P
            pltpu.sync_copy(idxc.at[pl.ds(r0, P)], idx_v)
            pltpu.async_copy(table.at[idx_v], rows_v, sem).wait()
            pltpu.sync_copy(rows_v, out.at[pl.ds(r0, P)])
            return carry

        lax.fori_loop(0, nch, chunk, 0)

    return kern


@functools.cache
def _lmax(C):
    return _make_lmax(C)


@functools.cache
def _gather():
    return _make_gather()


def _lin_body(m_ref, wl_ref, bl_ref, wc_ref, bc_ref, out_ref):
    t = jnp.dot(m_ref[...], wl_ref[...],
                preferred_element_type=jnp.float32) + bl_ref[...]
    out_ref[...] = jnp.maximum(
        jnp.dot(t, wc_ref[...], preferred_element_type=jnp.float32)
        + bc_ref[...], 0.0)


def _lin(m1, wl1T, bl1, wc1T, bc1):
    P = 2048
    M = B * N
    return pl.pallas_call(
        _lin_body,
        grid=(M // P,),
        in_specs=[
            pl.BlockSpec((P, 128), lambda i: (i, 0)),
            pl.BlockSpec((128, 64), lambda i: (0, 0)),
            pl.BlockSpec((1, 64), lambda i: (0, 0)),
            pl.BlockSpec((64, 128), lambda i: (0, 0)),
            pl.BlockSpec((1, 128), lambda i: (0, 0)),
        ],
        out_specs=pl.BlockSpec((P, 128), lambda i: (i, 0)),
        out_shape=jax.ShapeDtypeStruct((M, 128), jnp.float32),
    )(m1, wl1T, bl1, wc1T, bc1)


def _tail_body(m2_ref, wl2_ref, bl2_ref, wc2_ref, bc2_ref,
               wma_ref, bma_ref, wmb_ref, bmb_ref, out_ref):
    t = jnp.dot(m2_ref[0], wl2_ref[...],
                preferred_element_type=jnp.float32) + bl2_ref[...]
    u = jnp.dot(t, wc2_ref[...],
                preferred_element_type=jnp.float32) + bc2_ref[...]
    mx = jnp.max(u, axis=0, keepdims=True)
    a = jnp.maximum(
        jnp.dot(mx, wma_ref[...], preferred_element_type=jnp.float32)
        + bma_ref[...], 0.0)
    out_ref[0] = jnp.dot(a, wmb_ref[...],
                         preferred_element_type=jnp.float32) + bmb_ref[...]


def _tail(m2, wl2T, bl2, wc2T, bc2, wmaT, bma, wmbT, bmb):
    full = lambda b: (0, 0)
    return pl.pallas_call(
        _tail_body,
        grid=(B,),
        in_specs=[
            pl.BlockSpec((1, N, 128), lambda b: (b, 0, 0)),
            pl.BlockSpec((128, 128), full),
            pl.BlockSpec((1, 128), full),
            pl.BlockSpec((128, 1024), full),
            pl.BlockSpec((1, 1024), full),
            pl.BlockSpec((1024, 512), full),
            pl.BlockSpec((1, 512), full),
            pl.BlockSpec((512, 512), full),
            pl.BlockSpec((1, 512), full),
        ],
        out_specs=pl.BlockSpec((1, 1, 512), lambda b: (b, 0, 0)),
        out_shape=jax.ShapeDtypeStruct((B, 1, 512), jnp.float32),
    )(m2, wl2T, bl2, wc2T, bc2, wmaT, bma, wmbT, bmb)


def kernel(pts, W1a, b1a, W1b, b1b, W1c, b1c, Wl1, bl1, Wc1, bc1,
           Wl2, bl2, Wc2, bc2, Wm2a, bm2a, Wm2b, bm2b):
    w1aT = jnp.pad(W1a, ((0, 0), (0, 2))).T
    ptsT = jnp.transpose(pts, (0, 2, 1))
    idx = _knn(pts, ptsT)
    ptsf = pts.reshape(B * N, 2)
    pts_pad = jnp.pad(ptsf, ((0, 0), (0, 126)))
    g0 = _gather()(pts_pad, idx[:, :, 0].reshape(-1))
    g1 = _gather()(pts_pad, idx[:, :, 1].reshape(-1))
    f1 = _cov_mlp1(ptsf, g0, g1, w1aT, b1a[None, :], W1b.T, b1b[None, :],
                   W1c.T, b1c[None, :])
    idxf = idx.reshape(-1)
    wl1T = jnp.pad(Wl1.T, ((0, 64), (0, 0)))
    m1 = _lmax(128)(f1, idxf)
    f2 = _lin(m1, wl1T, bl1[None, :], Wc1.T, bc1[None, :])
    m2 = _lmax(128)(f2, idxf)
    feat = _tail(m2.reshape(B, N, 128), Wl2.T, bl2[None, :], Wc2.T,
                 bc2[None, :], Wm2a.T, bm2a[None, :], Wm2b.T, bm2b[None, :])
    return feat
